# baseline (speedup 1.0000x reference)
import functools

import numpy as np
import jax
import jax.numpy as jnp
from jax import lax
from jax.experimental import pallas as pl
from jax.experimental.pallas import tpu as pltpu
from jax.experimental.pallas import tpu_sc as plsc

_B = 64
_L_IMG = 196
_L_NLP = 512
_D = 768
_N_REMAIN_IMG = 49
_N_MASKED_IMG = 147
_NLP_REMAIN = 128

_NT = 32
_ICHUNK = 56
_NICHUNK = 56


def _sinusoidal_pe(max_len, d_model):
    position = np.arange(max_len, dtype=np.float64)[:, None]
    div_term = np.exp(np.arange(0, d_model, 2, dtype=np.float64) * (-np.log(10000.0) / d_model))
    pe = np.zeros((max_len, d_model), dtype=np.float64)
    pe[:, 0::2] = np.sin(position * div_term)
    pe[:, 1::2] = np.cos(position * div_term)
    return pe.astype(np.float32)


def _threefry2x32_np(k0, k1, x0, x1):
    def rotl(x, d):
        return ((x << np.uint32(d)) | (x >> np.uint32(32 - d))).astype(np.uint32)
    ks = [np.uint32(k0), np.uint32(k1), np.uint32(k0 ^ k1 ^ 0x1BD11BDA)]
    x0 = (x0 + ks[0]).astype(np.uint32)
    x1 = (x1 + ks[1]).astype(np.uint32)
    rot_a = (13, 15, 26, 6)
    rot_b = (17, 29, 16, 24)
    for i, rots in enumerate((rot_a, rot_b, rot_a, rot_b, rot_a)):
        for r in rots:
            x0 = (x0 + x1).astype(np.uint32)
            x1 = rotl(x1, r)
            x1 = (x1 ^ x0).astype(np.uint32)
        x0 = (x0 + ks[(i + 1) % 3]).astype(np.uint32)
        x1 = (x1 + ks[(i + 2) % 3] + np.uint32(i + 1)).astype(np.uint32)
    return x0, x1


def _np_uniform_threefry(seed, shape):
    size = int(np.prod(shape))
    i = np.arange(size, dtype=np.uint64)
    counts1 = (i >> np.uint64(32)).astype(np.uint32)
    counts2 = (i & np.uint64(0xFFFFFFFF)).astype(np.uint32)
    k0 = np.uint32((seed >> 32) & 0xFFFFFFFF)
    k1 = np.uint32(seed & 0xFFFFFFFF)
    o0, o1 = _threefry2x32_np(k0, k1, counts1, counts2)
    bits = (o0 ^ o1).astype(np.uint32)
    fl = ((bits >> np.uint32(9)) | np.uint32(0x3F800000)).view(np.float32)
    return (fl - np.float32(1.0)).reshape(shape)


_SHUF = np.argsort(_np_uniform_threefry(42, (_B, _L_IMG)),
                   axis=-1, kind="stable").astype(np.int32)
_REMAIN_IDX = _SHUF[:, :_N_REMAIN_IMG]
_MASKED_IDX = _SHUF[:, _N_REMAIN_IMG:]
_REVERT_IDX = np.argsort(_SHUF, axis=-1).astype(np.int32)

_IIDX_SRC = np.ascontiguousarray(
    (_REMAIN_IDX + np.arange(_B, dtype=np.int32)[:, None] * _L_IMG).reshape(-1))
_IIDX_POS = np.ascontiguousarray(_REMAIN_IDX.reshape(-1))
_PE_TAB = _sinusoidal_pe(_L_NLP, _D)

_ONES_REMAIN = np.ones((_B, _N_REMAIN_IMG), np.float32)
_ONES_REVERT = np.ones((_B, _L_IMG), np.float32)
_ONES_MASKED = np.ones((_B, _N_MASKED_IMG), np.float32)


@functools.lru_cache(maxsize=1)
def _build_sc_kernel():
  mesh = plsc.VectorSubcoreMesh(core_axis_name="c", subcore_axis_name="s")

  @functools.partial(
      pl.kernel,
      mesh=mesh,
      out_type=[
          jax.ShapeDtypeStruct((_B * _N_REMAIN_IMG, _D), jnp.float32),
          jax.ShapeDtypeStruct((_B * _NLP_REMAIN, _D), jnp.float32),
      ],
      scratch_types=[
          pltpu.VMEM((64, _D), jnp.float32),
          pltpu.VMEM((64, _D), jnp.float32),
          pltpu.VMEM((_ICHUNK,), jnp.int32),
          pltpu.VMEM((_ICHUNK,), jnp.int32),
          pltpu.VMEM((64,), jnp.int32),
          pltpu.VMEM((64,), jnp.int32),
          pltpu.SemaphoreType.DMA,
          pltpu.SemaphoreType.DMA,
      ],
  )
  def _sc_gather_add(img_flat, nlp_flat, pos_tab, pe_tab, nlp_idx, iidx_src, iidx_pos,
                     out_img, out_nlp, buf_a, buf_b, i_src_v, i_pos_v, n_raw_v, n_flat_v,
                     sem_a, sem_b):
      t = lax.axis_index("s") * 2 + lax.axis_index("c")

      def _acc(nrows):
          def body(i, carry):
              r = i // 48
              o = (i % 48) * 16
              buf_a[r, pl.ds(o, 16)] = buf_a[r, pl.ds(o, 16)] + buf_b[r, pl.ds(o, 16)]
              return carry
          lax.fori_loop(0, nrows * 48, body, 0)

      def _img_chunk(cid):
          start = pl.multiple_of(cid * _ICHUNK, 8)
          pltpu.sync_copy(iidx_src.at[pl.ds(start, _ICHUNK)], i_src_v)
          pltpu.sync_copy(iidx_pos.at[pl.ds(start, _ICHUNK)], i_pos_v)
          ca = pltpu.async_copy(img_flat.at[i_src_v], buf_a.at[pl.ds(0, _ICHUNK)], sem_a)
          cb = pltpu.async_copy(pos_tab.at[i_pos_v], buf_b.at[pl.ds(0, _ICHUNK)], sem_b)
          ca.wait()
          cb.wait()
          _acc(_ICHUNK)
          pltpu.sync_copy(buf_a.at[pl.ds(0, _ICHUNK)], out_img.at[pl.ds(start, _ICHUNK)])

      _img_chunk(t)

      @pl.when(t < _NICHUNK - _NT)
      def _():
          _img_chunk(t + _NT)

      for k in range(2):
          b = t * 2 + k
          for c in range(2):
              off = pl.multiple_of(b * _NLP_REMAIN + c * 64, 64)
              pltpu.sync_copy(nlp_idx.at[pl.ds(off, 64)], n_raw_v)
              for j in range(4):
                  n_flat_v[pl.ds(j * 16, 16)] = n_raw_v[pl.ds(j * 16, 16)] + b * _L_NLP
              ca = pltpu.async_copy(nlp_flat.at[n_flat_v], buf_a, sem_a)
              cb = pltpu.async_copy(pe_tab.at[n_raw_v], buf_b, sem_b)
              ca.wait()
              cb.wait()
              _acc(64)
              pltpu.sync_copy(buf_a, out_nlp.at[pl.ds(off, 64)])
  return _sc_gather_add


def kernel(img, nlp, pos_enc_2d, nlp_remain_idx, nlp_masked_idx, nlp_revert_idx):
    img_flat = img.reshape(_B * _L_IMG, _D)
    nlp_flat = nlp.reshape(_B * _L_NLP, _D)
    out_img, out_nlp = _build_sc_kernel()(
        img_flat, nlp_flat, pos_enc_2d,
        jnp.asarray(_PE_TAB), nlp_remain_idx.astype(jnp.int32).reshape(-1),
        jnp.asarray(_IIDX_SRC), jnp.asarray(_IIDX_POS))
    return (
        out_img.reshape(_B, _N_REMAIN_IMG, _D),
        out_nlp.reshape(_B, _NLP_REMAIN, _D),
        jnp.asarray(_MASKED_IDX),
        jnp.asarray(_REVERT_IDX),
        jnp.asarray(_ONES_REMAIN),
        jnp.asarray(_ONES_REVERT),
        jnp.asarray(_ONES_MASKED),
    )
